# Initial kernel scaffold; baseline (speedup 1.0000x reference)
import jax
import jax.numpy as jnp
from jax.experimental import pallas as pl


def kernel(features, rois):
    raise NotImplementedError("write your pallas kernel here")


# speedup vs baseline: 4.5951x; 4.5951x over previous
import jax
import jax.numpy as jnp
from jax import lax
from jax.experimental import pallas as pl

N, C, H, W = 2, 128, 56, 56
PH = PW = 7
G = 2
SCALE = 0.25
OFFSET = 0.5
K = 1000
KB = 8
GRID = K // KB


def _roi_kernel(rois_ref, fm_ref, out_ref):
    r = rois_ref[...]
    bk = r[:, 0].astype(jnp.int32)
    sw = r[:, 1] * SCALE - OFFSET
    sh = r[:, 2] * SCALE - OFFSET
    ew = r[:, 3] * SCALE - OFFSET
    eh = r[:, 4] * SCALE - OFFSET
    bin_h = (eh - sh) / PH
    bin_w = (ew - sw) / PW

    pyi = lax.broadcasted_iota(jnp.int32, (KB, PH, G), 1).astype(jnp.float32)
    sv = (lax.broadcasted_iota(jnp.int32, (KB, PH, G), 2).astype(jnp.float32)
          + 0.5) / G
    y = sh[:, None, None] + (pyi + sv) * bin_h[:, None, None]
    my = ((y >= -1.0) & (y <= H)).astype(jnp.float32)
    yc = jnp.clip(y, 0.0, H - 1)
    y0f = jnp.floor(yc)
    y0 = y0f.astype(jnp.int32)
    ly = yc - y0f
    hy = 1.0 - ly
    y1 = jnp.minimum(y0 + 1, H - 1)
    col0 = bk[:, None, None] * H + y0
    col1 = bk[:, None, None] * H + y1
    iota_nh = lax.broadcasted_iota(jnp.int32, (KB, PH, G, N * H), 3)
    inv = 1.0 / (G * G)
    a = jnp.where(col0[..., None] == iota_nh, (my * hy * inv)[..., None], 0.0)
    a = a + jnp.where(col1[..., None] == iota_nh, (my * ly * inv)[..., None], 0.0)
    a2 = a.sum(axis=2).reshape(KB * PH, N * H)

    pxi = lax.broadcasted_iota(jnp.int32, (KB, PW, G), 1).astype(jnp.float32)
    x = sw[:, None, None] + (pxi + sv) * bin_w[:, None, None]
    mx = ((x >= -1.0) & (x <= W)).astype(jnp.float32)
    xc = jnp.clip(x, 0.0, W - 1)
    x0f = jnp.floor(xc)
    x0 = x0f.astype(jnp.int32)
    lx = xc - x0f
    hx = 1.0 - lx
    x1 = jnp.minimum(x0 + 1, W - 1)
    iota_w = lax.broadcasted_iota(jnp.int32, (KB, PW, G, W), 3)
    b = jnp.where(x0[..., None] == iota_w, (mx * hx)[..., None], 0.0)
    b = b + jnp.where(x1[..., None] == iota_w, (mx * lx)[..., None], 0.0)
    bmat = b.sum(axis=2)

    p = jnp.dot(a2, fm_ref[...], preferred_element_type=jnp.float32)
    p3 = p.reshape(KB * PH, W, C)

    bb = jnp.broadcast_to(bmat[:, None, :, :], (KB, PH, PW, W)).reshape(
        KB * PH, PW, W)
    prod = p3[:, None, :, :] * bb[..., None]
    o = prod.sum(axis=2)
    o4 = o.reshape(KB, PH, PW, C)
    out_ref[...] = jnp.transpose(o4, (0, 3, 1, 2))


def kernel(features, rois):
    fm = jnp.transpose(features, (0, 2, 3, 1)).reshape(N * H, W * C)
    out = pl.pallas_call(
        _roi_kernel,
        grid=(GRID,),
        in_specs=[
            pl.BlockSpec((KB, 5), lambda i: (i, 0)),
            pl.BlockSpec((N * H, W * C), lambda i: (0, 0)),
        ],
        out_specs=pl.BlockSpec((KB, C, PH, PW), lambda i: (i, 0, 0, 0)),
        out_shape=jax.ShapeDtypeStruct((K, C, PH, PW), jnp.float32),
    )(rois, fm)
    return out
